# baseline (speedup 1.0000x reference)
import functools

import jax
import jax.numpy as jnp
from jax import lax
from jax.experimental import pallas as pl
from jax.experimental.pallas import tpu as pltpu
from jax.experimental.pallas import tpu_sc as plsc

N = 10000
NP = 10240
E = 320000
ESL = E + N
NC, NS, L = 2, 16, 16
NW = NC * NS
CB = 128
NBUF = 6
CHUNKS = 84
CE = CHUNKS * CB
EPAD = NW * CE
ROWS_PER = NP // NS
F = 16


def _proj_tail(h, as_w, ad_w, h_ref, asv_ref, adv_ref, cvec_ref):
    h_ref[...] = h
    asv = jnp.dot(h, as_w, preferred_element_type=jnp.float32)
    adv = jnp.dot(h, ad_w, preferred_element_type=jnp.float32)
    asv_ref[...] = asv
    adv_ref[...] = adv
    s = jnp.max(asv) + jnp.max(adv)
    c = jnp.maximum(s, 0.2 * s)
    cvec_ref[...] = jnp.full((L,), c, jnp.float32)


def _tc_front_body(x_ref, w_ref, as_ref, ad_ref,
                   h_ref, asv_ref, adv_ref, cvec_ref):
    h = jnp.dot(x_ref[...], w_ref[...], preferred_element_type=jnp.float32)
    h = jnp.concatenate([h, jnp.zeros((NP - N, F), jnp.float32)], axis=0)
    _proj_tail(h, as_ref[...], ad_ref[...], h_ref, asv_ref, adv_ref, cvec_ref)


def _merge(num_ref, den_ref, b_ref):
    num = num_ref[0] + num_ref[1]
    den = den_ref[0] + den_ref[1]
    agg = num / (den[:, None] + 1e-16)
    g = agg + b_ref[...][None, :]
    return jnp.maximum(g, 0.2 * g)


def _tc_mid_body(num_ref, den_ref, b_ref, w_ref, as_ref, ad_ref,
                 h_ref, asv_ref, adv_ref, cvec_ref):
    g = _merge(num_ref, den_ref, b_ref)
    h = jnp.dot(g, w_ref[...], preferred_element_type=jnp.float32)
    _proj_tail(h, as_ref[...], ad_ref[...], h_ref, asv_ref, adv_ref, cvec_ref)


def _tc_final_body(num_ref, den_ref, b_ref,
                   w1_ref, b1_ref, w2_ref, b2_ref, w3_ref, b3_ref,
                   w4_ref, b4_ref, out_ref):
    g = _merge(num_ref, den_ref, b_ref)[:N]
    t = jnp.dot(g, w1_ref[...], preferred_element_type=jnp.float32) + b1_ref[...][None, :]
    t = jnp.maximum(t, 0.0)
    t = jnp.dot(t, w2_ref[...], preferred_element_type=jnp.float32) + b2_ref[...][None, :]
    t = jnp.maximum(t, 0.0)
    t = jnp.dot(t, w3_ref[...], preferred_element_type=jnp.float32) + b3_ref[...][None, :]
    t = jnp.maximum(t, 0.0)
    out_ref[...] = jnp.dot(t, w4_ref[...], preferred_element_type=jnp.float32) + b4_ref[...][None, :]


_TC_OUT = (
    jax.ShapeDtypeStruct((NP, F), jnp.float32),
    jax.ShapeDtypeStruct((NP,), jnp.float32),
    jax.ShapeDtypeStruct((NP,), jnp.float32),
    jax.ShapeDtypeStruct((L,), jnp.float32),
)


def _tc_front(x_pad, W, a_s, a_d):
    return pl.pallas_call(_tc_front_body, out_shape=_TC_OUT)(x_pad, W, a_s, a_d)


def _tc_mid(numP, denP, b, W, a_s, a_d):
    return pl.pallas_call(_tc_mid_body, out_shape=_TC_OUT)(
        numP, denP, b, W, a_s, a_d)


def _tc_final(numP, denP, b, Wl1, bl1, Wl2, bl2, Wl3, bl3, Wl4, bl4):
    return pl.pallas_call(
        _tc_final_body,
        out_shape=jax.ShapeDtypeStruct((N, 128), jnp.float32),
    )(numP, denP, b, Wl1, bl1, Wl2, bl2, Wl3, bl3, Wl4, bl4)


def _sc_edges_body(idx_hbm, h_hbm, as_hbm, ad_hbm, cvec_hbm,
                   num_out, den_out,
                   as_v, ad_v, c_v, idx_v, hrs, wrs, pbs,
                   num_sh, den_sh, gsems, nsems, dsems):
    cid = lax.axis_index("c")
    sid = lax.axis_index("s")
    wid = cid * NS + sid

    pltpu.sync_copy(as_hbm, as_v)
    pltpu.sync_copy(ad_hbm, ad_v)
    pltpu.sync_copy(cvec_hbm, c_v)
    pltpu.sync_copy(idx_hbm.at[wid], idx_v)
    cval = c_v[...]

    zero16 = jnp.zeros((L,), jnp.float32)

    def _zrow(j, carry):
        wrs[0][j, :] = zero16
        pbs[0][pl.ds((j % 8) * L, L)] = zero16
        return carry

    lax.fori_loop(0, CB, _zrow, 0)
    row0 = sid * ROWS_PER
    for k in range(ROWS_PER // CB):
        pltpu.sync_copy(wrs[0], num_sh.at[pl.ds(row0 + k * CB, CB)])
        pltpu.sync_copy(pbs[0], den_sh.at[pl.ds(row0 + k * CB, CB)])
    plsc.subcore_barrier()

    def _compute(k, hr, wr, pb):
        for g in range(CB // L):
            siv = idx_v[k, pl.ds(g * L, L)]
            div = idx_v[CHUNKS + k, pl.ds(g * L, L)]
            av = plsc.load_gather(as_v, [siv])
            dv = plsc.load_gather(ad_v, [div])
            s = av + dv
            e = jnp.maximum(s, 0.2 * s)
            p = jnp.exp(e - cval)
            pb[pl.ds(g * L, L)] = p
            for j in range(L):
                pi = p[j]
                wr[g * L + j, :] = hr[g * L + j, :] * pi

    for b in range(NBUF):
        pltpu.async_copy(h_hbm.at[idx_v.at[b]], hrs[b], gsems[b])

    def _round(i, carry):
        for b in range(NBUF):
            k = NBUF * i + b
            pltpu.make_async_copy(h_hbm.at[idx_v.at[k]], hrs[b],
                                  gsems[b]).wait()

            @pl.when(i > 0)
            def _():
                dk = idx_v.at[CHUNKS + k - NBUF]
                pltpu.make_async_copy(wrs[b], num_sh.at[dk], nsems[b]).wait()
                pltpu.make_async_copy(pbs[b], den_sh.at[dk], dsems[b]).wait()

            _compute(k, hrs[b], wrs[b], pbs[b])
            dk = idx_v.at[CHUNKS + k]
            pltpu.async_copy(wrs[b], num_sh.at[dk], nsems[b], add=True)
            pltpu.async_copy(pbs[b], den_sh.at[dk], dsems[b], add=True)

            @pl.when(k + NBUF < CHUNKS)
            def _():
                pltpu.async_copy(h_hbm.at[idx_v.at[k + NBUF]], hrs[b],
                                 gsems[b])
        return carry

    lax.fori_loop(0, CHUNKS // NBUF, _round, 0)
    for b in range(NBUF):
        dk = idx_v.at[2 * CHUNKS - NBUF + b]
        pltpu.make_async_copy(wrs[b], num_sh.at[dk], nsems[b]).wait()
        pltpu.make_async_copy(pbs[b], den_sh.at[dk], dsems[b]).wait()
    plsc.subcore_barrier()

    for k in range(ROWS_PER // CB):
        r = row0 + k * CB
        pltpu.sync_copy(num_sh.at[pl.ds(r, CB)], wrs[0])
        pltpu.sync_copy(wrs[0], num_out.at[cid].at[pl.ds(r, CB)])
        pltpu.sync_copy(den_sh.at[pl.ds(r, CB)], pbs[0])
        pltpu.sync_copy(pbs[0], den_out.at[cid].at[pl.ds(r, CB)])


_sc_edges = pl.kernel(
    _sc_edges_body,
    out_type=(
        jax.ShapeDtypeStruct((NC, NP, F), jnp.float32),
        jax.ShapeDtypeStruct((NC, NP), jnp.float32),
    ),
    mesh=plsc.VectorSubcoreMesh(core_axis_name="c", subcore_axis_name="s",
                                num_cores=NC, num_subcores=NS),
    scratch_types=[
        pltpu.VMEM((NP,), jnp.float32),
        pltpu.VMEM((NP,), jnp.float32),
        pltpu.VMEM((L,), jnp.float32),
        pltpu.VMEM((2 * CHUNKS, CB), jnp.int32),
        tuple(pltpu.VMEM((CB, F), jnp.float32) for _ in range(NBUF)),
        tuple(pltpu.VMEM((CB, F), jnp.float32) for _ in range(NBUF)),
        tuple(pltpu.VMEM((CB,), jnp.float32) for _ in range(NBUF)),
        pltpu.VMEM_SHARED((NP, F), jnp.float32),
        pltpu.VMEM_SHARED((NP,), jnp.float32),
        tuple(pltpu.SemaphoreType.DMA for _ in range(NBUF)),
        tuple(pltpu.SemaphoreType.DMA for _ in range(NBUF)),
        tuple(pltpu.SemaphoreType.DMA for _ in range(NBUF)),
    ],
    compiler_params=pltpu.CompilerParams(needs_layout_passes=False,
                                         use_tc_tiling_on_sc=False),
)


def kernel(x, edge_index, W1, a1s, a1d, b1, W2, a2s, a2d, b2,
           W3, a3s, a3d, b3, Wl1, bl1, Wl2, bl2, Wl3, bl3, Wl4, bl4):
    loop = jnp.arange(N, dtype=jnp.int32)
    padi = N + jnp.arange(EPAD - ESL, dtype=jnp.int32) % (NP - N)
    srcp = jnp.concatenate([edge_index[0], loop, padi]).reshape(NW, CHUNKS, CB)
    dstp = jnp.concatenate([edge_index[1], loop, padi]).reshape(NW, CHUNKS, CB)
    idx3 = jnp.concatenate([srcp, dstp], axis=1)

    h, asv, adv, cvec = _tc_front(x, W1, a1s, a1d)
    numP, denP = _sc_edges(idx3, h, asv, adv, cvec)
    h, asv, adv, cvec = _tc_mid(numP, denP, b1, W2, a2s, a2d)
    numP, denP = _sc_edges(idx3, h, asv, adv, cvec)
    h, asv, adv, cvec = _tc_mid(numP, denP, b2, W3, a3s, a3d)
    numP, denP = _sc_edges(idx3, h, asv, adv, cvec)
    return _tc_final(numP, denP, b3, Wl1, bl1, Wl2, bl2, Wl3, bl3, Wl4, bl4)


# speedup vs baseline: 1.1448x; 1.1448x over previous
import functools

import jax
import jax.numpy as jnp
from jax import lax
from jax.experimental import pallas as pl
from jax.experimental.pallas import tpu as pltpu
from jax.experimental.pallas import tpu_sc as plsc

N = 10000
NP = 10240
E = 320000
ESL = E + N
NC, NS, L = 2, 16, 16
NW = NC * NS
CB = 128
NBUF = 4
CHUNKS = 84
CE = CHUNKS * CB
EPAD = NW * CE
ROWS_PER = NP // NS
F = 16


def _proj_tail(h, as_w, ad_w, h_ref, asv_ref, adv_ref, cvec_ref):
    h_ref[...] = h
    asv = jnp.dot(h, as_w, preferred_element_type=jnp.float32)
    adv = jnp.dot(h, ad_w, preferred_element_type=jnp.float32)
    asv_ref[...] = asv
    adv_ref[...] = adv
    s = jnp.max(asv) + jnp.max(adv)
    c = jnp.maximum(s, 0.2 * s)
    cvec_ref[...] = jnp.full((L,), c, jnp.float32)


def _tc_front_body(x_ref, w_ref, as_ref, ad_ref,
                   h_ref, asv_ref, adv_ref, cvec_ref):
    h = jnp.dot(x_ref[...], w_ref[...], preferred_element_type=jnp.float32)
    h = jnp.concatenate([h, jnp.zeros((NP - N, F), jnp.float32)], axis=0)
    _proj_tail(h, as_ref[...], ad_ref[...], h_ref, asv_ref, adv_ref, cvec_ref)


def _merge(num_ref, den_ref, b_ref):
    num = num_ref[0] + num_ref[1]
    den = den_ref[0] + den_ref[1]
    agg = num / (den[:, None] + 1e-16)
    g = agg + b_ref[...][None, :]
    return jnp.maximum(g, 0.2 * g)


def _tc_mid_body(num_ref, den_ref, b_ref, w_ref, as_ref, ad_ref,
                 h_ref, asv_ref, adv_ref, cvec_ref):
    g = _merge(num_ref, den_ref, b_ref)
    h = jnp.dot(g, w_ref[...], preferred_element_type=jnp.float32)
    _proj_tail(h, as_ref[...], ad_ref[...], h_ref, asv_ref, adv_ref, cvec_ref)


def _tc_final_body(num_ref, den_ref, b_ref,
                   w1_ref, b1_ref, w2_ref, b2_ref, w3_ref, b3_ref,
                   w4_ref, b4_ref, out_ref):
    g = _merge(num_ref, den_ref, b_ref)[:N]
    t = jnp.dot(g, w1_ref[...], preferred_element_type=jnp.float32) + b1_ref[...][None, :]
    t = jnp.maximum(t, 0.0)
    t = jnp.dot(t, w2_ref[...], preferred_element_type=jnp.float32) + b2_ref[...][None, :]
    t = jnp.maximum(t, 0.0)
    t = jnp.dot(t, w3_ref[...], preferred_element_type=jnp.float32) + b3_ref[...][None, :]
    t = jnp.maximum(t, 0.0)
    out_ref[...] = jnp.dot(t, w4_ref[...], preferred_element_type=jnp.float32) + b4_ref[...][None, :]


_TC_OUT = (
    jax.ShapeDtypeStruct((NP, F), jnp.float32),
    jax.ShapeDtypeStruct((NP,), jnp.float32),
    jax.ShapeDtypeStruct((NP,), jnp.float32),
    jax.ShapeDtypeStruct((L,), jnp.float32),
)


def _tc_front(x_pad, W, a_s, a_d):
    return pl.pallas_call(_tc_front_body, out_shape=_TC_OUT)(x_pad, W, a_s, a_d)


def _tc_mid(numP, denP, b, W, a_s, a_d):
    return pl.pallas_call(_tc_mid_body, out_shape=_TC_OUT)(
        numP, denP, b, W, a_s, a_d)


def _tc_final(numP, denP, b, Wl1, bl1, Wl2, bl2, Wl3, bl3, Wl4, bl4):
    return pl.pallas_call(
        _tc_final_body,
        out_shape=jax.ShapeDtypeStruct((N, 128), jnp.float32),
    )(numP, denP, b, Wl1, bl1, Wl2, bl2, Wl3, bl3, Wl4, bl4)


def _sc_edges_body(idx_hbm, h_hbm, as_hbm, ad_hbm, cvec_hbm,
                   num_out, den_out,
                   as_v, ad_v, c_v, idx_v, hrs, wrs, pbs,
                   num_sh, den_sh, gsems, nsems, dsems):
    cid = lax.axis_index("c")
    sid = lax.axis_index("s")
    wid = cid * NS + sid

    pltpu.sync_copy(as_hbm, as_v)
    pltpu.sync_copy(ad_hbm, ad_v)
    pltpu.sync_copy(cvec_hbm, c_v)
    pltpu.sync_copy(idx_hbm.at[wid], idx_v)
    cval = c_v[...]

    zero16 = jnp.zeros((L,), jnp.float32)

    def _zrow(j, carry):
        wrs[0][j, :] = zero16
        pbs[0][pl.ds((j % 8) * L, L)] = zero16
        return carry

    lax.fori_loop(0, CB, _zrow, 0)
    row0 = sid * ROWS_PER
    for k in range(ROWS_PER // CB):
        pltpu.sync_copy(wrs[0], num_sh.at[pl.ds(row0 + k * CB, CB)])
        pltpu.sync_copy(pbs[0], den_sh.at[pl.ds(row0 + k * CB, CB)])
    plsc.subcore_barrier()

    def _compute(k, hr, wr, pb):
        for g in range(CB // L):
            siv = idx_v[k, pl.ds(g * L, L)]
            div = idx_v[CHUNKS + k, pl.ds(g * L, L)]
            av = plsc.load_gather(as_v, [siv])
            dv = plsc.load_gather(ad_v, [div])
            s = av + dv
            e = jnp.maximum(s, 0.2 * s)
            p = jnp.exp(e - cval)
            pb[pl.ds(g * L, L)] = p
            for j in range(L):
                pi = p[j]
                wr[g * L + j, :] = hr[g * L + j, :] * pi

    for b in range(NBUF):
        pltpu.async_copy(h_hbm.at[idx_v.at[b]], hrs[b], gsems[b])

    def _round(i, carry):
        for b in range(NBUF):
            k = NBUF * i + b
            pltpu.make_async_copy(h_hbm.at[idx_v.at[k]], hrs[b],
                                  gsems[b]).wait()

            @pl.when(i > 0)
            def _():
                dk = idx_v.at[CHUNKS + k - NBUF]
                pltpu.make_async_copy(wrs[b], num_sh.at[dk], nsems[b]).wait()
                pltpu.make_async_copy(pbs[b], den_sh.at[dk], dsems[b]).wait()

            _compute(k, hrs[b], wrs[b], pbs[b])
            dk = idx_v.at[CHUNKS + k]
            pltpu.async_copy(wrs[b], num_sh.at[dk], nsems[b], add=True)
            pltpu.async_copy(pbs[b], den_sh.at[dk], dsems[b], add=True)

            @pl.when(k + NBUF < CHUNKS)
            def _():
                pltpu.async_copy(h_hbm.at[idx_v.at[k + NBUF]], hrs[b],
                                 gsems[b])
        return carry

    lax.fori_loop(0, CHUNKS // NBUF, _round, 0)
    for b in range(NBUF):
        dk = idx_v.at[2 * CHUNKS - NBUF + b]
        pltpu.make_async_copy(wrs[b], num_sh.at[dk], nsems[b]).wait()
        pltpu.make_async_copy(pbs[b], den_sh.at[dk], dsems[b]).wait()
    plsc.subcore_barrier()

    for k in range(ROWS_PER // CB):
        r = row0 + k * CB
        pltpu.sync_copy(num_sh.at[pl.ds(r, CB)], wrs[0])
        pltpu.sync_copy(wrs[0], num_out.at[cid].at[pl.ds(r, CB)])
        pltpu.sync_copy(den_sh.at[pl.ds(r, CB)], pbs[0])
        pltpu.sync_copy(pbs[0], den_out.at[cid].at[pl.ds(r, CB)])


_sc_edges = pl.kernel(
    _sc_edges_body,
    out_type=(
        jax.ShapeDtypeStruct((NC, NP, F), jnp.float32),
        jax.ShapeDtypeStruct((NC, NP), jnp.float32),
    ),
    mesh=plsc.VectorSubcoreMesh(core_axis_name="c", subcore_axis_name="s",
                                num_cores=NC, num_subcores=NS),
    scratch_types=[
        pltpu.VMEM((NP,), jnp.float32),
        pltpu.VMEM((NP,), jnp.float32),
        pltpu.VMEM((L,), jnp.float32),
        pltpu.VMEM((2 * CHUNKS, CB), jnp.int32),
        tuple(pltpu.VMEM((CB, F), jnp.float32) for _ in range(NBUF)),
        tuple(pltpu.VMEM((CB, F), jnp.float32) for _ in range(NBUF)),
        tuple(pltpu.VMEM((CB,), jnp.float32) for _ in range(NBUF)),
        pltpu.VMEM_SHARED((NP, F), jnp.float32),
        pltpu.VMEM_SHARED((NP,), jnp.float32),
        tuple(pltpu.SemaphoreType.DMA for _ in range(NBUF)),
        tuple(pltpu.SemaphoreType.DMA for _ in range(NBUF)),
        tuple(pltpu.SemaphoreType.DMA for _ in range(NBUF)),
    ],
    compiler_params=pltpu.CompilerParams(needs_layout_passes=False,
                                         use_tc_tiling_on_sc=False),
)


def kernel(x, edge_index, W1, a1s, a1d, b1, W2, a2s, a2d, b2,
           W3, a3s, a3d, b3, Wl1, bl1, Wl2, bl2, Wl3, bl3, Wl4, bl4):
    loop = jnp.arange(N, dtype=jnp.int32)
    padi = N + jnp.arange(EPAD - ESL, dtype=jnp.int32) % (NP - N)
    srcp = jnp.concatenate([edge_index[0], loop, padi]).reshape(NW, CHUNKS, CB)
    dstp = jnp.concatenate([edge_index[1], loop, padi]).reshape(NW, CHUNKS, CB)
    idx3 = jnp.concatenate([srcp, dstp], axis=1)

    h, asv, adv, cvec = _tc_front(x, W1, a1s, a1d)
    numP, denP = _sc_edges(idx3, h, asv, adv, cvec)
    h, asv, adv, cvec = _tc_mid(numP, denP, b1, W2, a2s, a2d)
    numP, denP = _sc_edges(idx3, h, asv, adv, cvec)
    h, asv, adv, cvec = _tc_mid(numP, denP, b2, W3, a3s, a3d)
    numP, denP = _sc_edges(idx3, h, asv, adv, cvec)
    return _tc_final(numP, denP, b3, Wl1, bl1, Wl2, bl2, Wl3, bl3, Wl4, bl4)
